# baseline (speedup 1.0000x reference)
import functools

import jax
import jax.numpy as jnp
from jax import lax
from jax.experimental import pallas as pl
from jax.experimental.pallas import tpu as pltpu
from jax.experimental.pallas import tpu_sc as plsc

B = 4
H = 32
S = 4096
D = 2048
K = 1024

_IB = 512
_NIB = S // _IB

def _topk_idx_kernel(aw_ref, idx_ref):
    b = pl.program_id(0)
    aw = aw_ref[0]

    t = aw[0:8] + aw[8:16] + aw[16:24] + aw[24:32]
    u = t[0:4] + t[4:8]
    v = u[0:2] + u[2:4]
    srow = (v[0:1] + v[1:2]) * jnp.float32(1.0 / 32.0)

    scol = jnp.reshape(srow, (S, 1))

    rank_col = [None] * _NIB
    row_acc = [None] * _NIB
    for ib in range(_NIB):
        sc = lax.slice(scol, (ib * _IB, 0), (ib * _IB + _IB, 1))
        icol = lax.broadcasted_iota(jnp.int32, (_IB, 1), 0) + ib * _IB
        acc = None
        for jb in range(ib, _NIB):
            sr = lax.slice(srow, (0, jb * _IB), (1, jb * _IB + _IB))
            if jb == ib:
                jrow = (lax.broadcasted_iota(jnp.int32, (1, _IB), 1)
                        + jb * _IB)
                tie = (sr == sc) & (jrow < icol)
                c = jnp.where((sr > sc) | tie, 1.0, 0.0)
            else:
                gt = sr > sc
                c = jnp.where(gt, 1.0, 0.0)
                cs = jnp.float32(_IB) - jnp.sum(c, axis=0, keepdims=True)
                row_acc[jb] = cs if row_acc[jb] is None else row_acc[jb] + cs
            r = jnp.sum(c, axis=1, keepdims=True)
            acc = r if acc is None else acc + r
        rank_col[ib] = acc

    zero_row = jnp.zeros((1, _IB), jnp.float32)
    row_full = lax.concatenate(
        [row_acc[jb] if row_acc[jb] is not None else zero_row
         for jb in range(_NIB)], 1)
    row_as_col = jnp.reshape(row_full, (S, 1))

    rrow = lax.broadcasted_iota(jnp.int32, (1, K), 1).astype(jnp.float32)
    racc = jnp.zeros((1, K), jnp.float32)
    for ib in range(_NIB):
        rank = rank_col[ib] + lax.slice(
            row_as_col, (ib * _IB, 0), (ib * _IB + _IB, 1))
        hit = rank == rrow
        icolf = (lax.broadcasted_iota(jnp.int32, (_IB, 1), 0)
                 + ib * _IB).astype(jnp.float32)
        racc = racc + jnp.sum(jnp.where(hit, icolf, 0.0), axis=0,
                              keepdims=True)
    idx_ref[0] = racc.astype(jnp.int32) + b * S


def _compute_topk_indices(attention_weights):
    out = pl.pallas_call(
        _topk_idx_kernel,
        grid=(B,),
        in_specs=[pl.BlockSpec((1, H, S), lambda b: (b, 0, 0))],
        out_specs=pl.BlockSpec((1, 1, K), lambda b: (b, 0, 0)),
        out_shape=jax.ShapeDtypeStruct((B, 1, K), jnp.int32),
    )(attention_weights)
    return out.reshape(B * K)


_NC = 2
_NS = 16
_NW = _NC * _NS
_RPW = (B * K) // _NW
_CH = 16
_NCH = _RPW // _CH


def _sc_gather_body(table_hbm, idx_hbm, out_hbm,
                    idx_v, buf0, buf1, g0, g1, s0, s1):
    wid = lax.axis_index("s") * _NC + lax.axis_index("c")
    base = wid * _RPW
    pltpu.sync_copy(idx_hbm.at[pl.ds(base, _RPW)], idx_v)
    bufs = (buf0, buf1)
    gsems = (g0, g1)
    ssems = (s0, s1)

    def start_gather(c):
        return pltpu.async_copy(
            table_hbm.at[idx_v.at[pl.ds(c * _CH, _CH)]],
            bufs[c % 2], gsems[c % 2])

    def start_write(c):
        return pltpu.async_copy(
            bufs[c % 2], out_hbm.at[pl.ds(base + c * _CH, _CH)],
            ssems[c % 2])

    gh = [None] * _NCH
    sh = [None] * _NCH
    gh[0] = start_gather(0)
    for c in range(_NCH):
        if c + 1 < _NCH:
            if c >= 1:
                sh[c - 1].wait()
            gh[c + 1] = start_gather(c + 1)
        gh[c].wait()
        sh[c] = start_write(c)
    sh[_NCH - 2].wait()
    sh[_NCH - 1].wait()


def _sc_gather(table, idx):
    mesh = plsc.VectorSubcoreMesh(core_axis_name="c", subcore_axis_name="s")
    run = functools.partial(
        pl.kernel,
        out_type=jax.ShapeDtypeStruct((B * K, D), jnp.float32),
        mesh=mesh,
        scratch_types=[
            pltpu.VMEM((_RPW,), jnp.int32),
            pltpu.VMEM((_CH, D), jnp.float32),
            pltpu.VMEM((_CH, D), jnp.float32),
            pltpu.SemaphoreType.DMA,
            pltpu.SemaphoreType.DMA,
            pltpu.SemaphoreType.DMA,
            pltpu.SemaphoreType.DMA,
        ],
    )(_sc_gather_body)
    return run(table, idx)


def kernel(x, attention_weights, head_weights):
    del head_weights
    idx = _compute_topk_indices(attention_weights)
    table = x.reshape(B * S, D)
    out = _sc_gather(table, idx)
    return out.reshape(B, K, D)


# speedup vs baseline: 1.0135x; 1.0135x over previous
import functools

import jax
import jax.numpy as jnp
from jax import lax
from jax.experimental import pallas as pl
from jax.experimental.pallas import tpu as pltpu
from jax.experimental.pallas import tpu_sc as plsc

B = 4
H = 32
S = 4096
D = 2048
K = 1024

_IB = 512
_NIB = S // _IB

def _topk_idx_kernel(aw_ref, idx_ref):
    b = pl.program_id(0)
    aw = aw_ref[0]

    t = aw[0:8] + aw[8:16] + aw[16:24] + aw[24:32]
    u = t[0:4] + t[4:8]
    v = u[0:2] + u[2:4]
    srow = (v[0:1] + v[1:2]) * jnp.float32(1.0 / 32.0)

    scol = jnp.reshape(srow, (S, 1))

    rank_col = [None] * _NIB
    row_acc = [None] * _NIB
    for ib in range(_NIB):
        sc = lax.slice(scol, (ib * _IB, 0), (ib * _IB + _IB, 1))
        icol = lax.broadcasted_iota(jnp.int32, (_IB, 1), 0) + ib * _IB
        acc = None
        for jb in range(ib, _NIB):
            sr = lax.slice(srow, (0, jb * _IB), (1, jb * _IB + _IB))
            if jb == ib:
                jrow = (lax.broadcasted_iota(jnp.int32, (1, _IB), 1)
                        + jb * _IB)
                tie = (sr == sc) & (jrow < icol)
                c = jnp.where((sr > sc) | tie, 1.0, 0.0)
            else:
                gt = sr > sc
                c = jnp.where(gt, 1.0, 0.0)
                cs = jnp.float32(_IB) - jnp.sum(c, axis=0, keepdims=True)
                row_acc[jb] = cs if row_acc[jb] is None else row_acc[jb] + cs
            r = jnp.sum(c, axis=1, keepdims=True)
            acc = r if acc is None else acc + r
        rank_col[ib] = acc

    zero_row = jnp.zeros((1, _IB), jnp.float32)
    row_full = lax.concatenate(
        [row_acc[jb] if row_acc[jb] is not None else zero_row
         for jb in range(_NIB)], 1)
    row_as_col = jnp.reshape(row_full, (S, 1))

    rrow = lax.broadcasted_iota(jnp.int32, (1, K), 1).astype(jnp.float32)
    racc = jnp.zeros((1, K), jnp.float32)
    for ib in range(_NIB):
        rank = rank_col[ib] + lax.slice(
            row_as_col, (ib * _IB, 0), (ib * _IB + _IB, 1))
        hit = rank == rrow
        icolf = (lax.broadcasted_iota(jnp.int32, (_IB, 1), 0)
                 + ib * _IB).astype(jnp.float32)
        racc = racc + jnp.sum(jnp.where(hit, icolf, 0.0), axis=0,
                              keepdims=True)
    idx_ref[0] = racc.astype(jnp.int32) + b * S


def _compute_topk_indices(attention_weights):
    out = pl.pallas_call(
        _topk_idx_kernel,
        grid=(B,),
        in_specs=[pl.BlockSpec((1, H, S), lambda b: (b, 0, 0))],
        out_specs=pl.BlockSpec((1, 1, K), lambda b: (b, 0, 0)),
        out_shape=jax.ShapeDtypeStruct((B, 1, K), jnp.int32),
    )(attention_weights)
    return out.reshape(B * K)


_NC = 2
_NS = 16
_NW = _NC * _NS
_RPW = (B * K) // _NW
_CH = 16
_NCH = _RPW // _CH


_NBUF = 3


def _sc_gather_body(table_hbm, idx_hbm, out_hbm,
                    idx_v, buf0, buf1, buf2, g0, g1, g2, s0, s1, s2):
    wid = lax.axis_index("s") * _NC + lax.axis_index("c")
    base = wid * _RPW
    pltpu.sync_copy(idx_hbm.at[pl.ds(base, _RPW)], idx_v)
    bufs = (buf0, buf1, buf2)
    gsems = (g0, g1, g2)
    ssems = (s0, s1, s2)

    def start_gather(c):
        return pltpu.async_copy(
            table_hbm.at[idx_v.at[pl.ds(c * _CH, _CH)]],
            bufs[c % _NBUF], gsems[c % _NBUF])

    def start_write(c):
        return pltpu.async_copy(
            bufs[c % _NBUF], out_hbm.at[pl.ds(base + c * _CH, _CH)],
            ssems[c % _NBUF])

    gh = [None] * _NCH
    sh = [None] * _NCH
    gh[0] = start_gather(0)
    gh[1] = start_gather(1)
    for c in range(_NCH):
        if c + 2 < _NCH:
            if c >= 1:
                sh[c - 1].wait()
            gh[c + 2] = start_gather(c + 2)
        gh[c].wait()
        sh[c] = start_write(c)
    sh[_NCH - 3].wait()
    sh[_NCH - 2].wait()
    sh[_NCH - 1].wait()


def _sc_gather(table, idx):
    mesh = plsc.VectorSubcoreMesh(core_axis_name="c", subcore_axis_name="s")
    run = functools.partial(
        pl.kernel,
        out_type=jax.ShapeDtypeStruct((B * K, D), jnp.float32),
        mesh=mesh,
        scratch_types=[
            pltpu.VMEM((_RPW,), jnp.int32),
            pltpu.VMEM((_CH, D), jnp.float32),
            pltpu.VMEM((_CH, D), jnp.float32),
            pltpu.VMEM((_CH, D), jnp.float32),
            pltpu.SemaphoreType.DMA,
            pltpu.SemaphoreType.DMA,
            pltpu.SemaphoreType.DMA,
            pltpu.SemaphoreType.DMA,
            pltpu.SemaphoreType.DMA,
            pltpu.SemaphoreType.DMA,
        ],
    )(_sc_gather_body)
    return run(table, idx)


def kernel(x, attention_weights, head_weights):
    del head_weights
    idx = _compute_topk_indices(attention_weights)
    table = x.reshape(B * S, D)
    out = _sc_gather(table, idx)
    return out.reshape(B, K, D)
